# baseline (speedup 1.0000x reference)
import functools

import jax
import jax.numpy as jnp
from jax import lax
from jax.experimental import pallas as pl
from jax.experimental.pallas import tpu as pltpu
from jax.experimental.pallas import tpu_sc as plsc

_FEAT_DIM = 64
_NUM_CLASSES = 100000
_BATCH = 16384
_LAMBDA_C = 1.0

_NC = 2
_NS = 16
_L = 16
_NW = _NC * _NS
_B_PER_W = _BATCH // _NW
_N_PASS = 2
_B_PASS = _B_PER_W // _N_PASS
_IDX_CHUNK = 128
_C_BLK = 128
_N_BLK = (_NUM_CLASSES + _C_BLK - 1) // _C_BLK
_P_ROWS = _N_BLK * (_C_BLK // 2)


def _pack_body(ct_ref, out_ref):
    q = lax.broadcasted_iota(jnp.int32, (_C_BLK // 2, _C_BLK), 0)
    k = lax.broadcasted_iota(jnp.int32, (_C_BLK // 2, _C_BLK), 1)
    e_even = (k == 2 * q).astype(jnp.float32)
    e_odd = (k == 2 * q + 1).astype(jnp.float32)
    ct = ct_ref[...]
    dn = (((1,), (1,)), ((), ()))
    out_ref[:, 0:_FEAT_DIM] = lax.dot_general(
        e_even, ct, dn, preferred_element_type=jnp.float32)
    out_ref[:, _FEAT_DIM:] = lax.dot_general(
        e_odd, ct, dn, preferred_element_type=jnp.float32)


def _pack_pairs(centers_t):
    return pl.pallas_call(
        _pack_body,
        grid=(_N_BLK,),
        in_specs=[pl.BlockSpec((_FEAT_DIM, _C_BLK), lambda b: (0, b))],
        out_specs=pl.BlockSpec((_C_BLK // 2, _C_BLK), lambda b: (b, 0)),
        out_shape=jax.ShapeDtypeStruct((_P_ROWS, _C_BLK), jnp.float32),
    )(centers_t)


def _sc_partials(feat, label, ptable):
    mesh = plsc.VectorSubcoreMesh(core_axis_name="c", subcore_axis_name="s")

    @functools.partial(
        pl.kernel,
        mesh=mesh,
        out_type=jax.ShapeDtypeStruct((_NW, _L), jnp.float32),
        scratch_types=[
            pltpu.VMEM((_B_PER_W,), jnp.int32),
            pltpu.VMEM((_B_PER_W // _IDX_CHUNK, _IDX_CHUNK), jnp.int32),
            pltpu.VMEM((_B_PASS, _FEAT_DIM), jnp.float32),
            pltpu.VMEM((_B_PASS, _C_BLK), jnp.float32),
            pltpu.VMEM((_L,), jnp.float32),
            pltpu.SemaphoreType.DMA,
        ],
        compiler_params=pltpu.CompilerParams(use_tc_tiling_on_sc=True),
    )
    def k(feat_hbm, label_hbm, ptable_hbm, out_hbm,
          lab_v, idx_v, feat_v, cent_v, acc_v, sem):
        wid = lax.axis_index("s") * _NC + lax.axis_index("c")
        base = wid * _B_PER_W
        pltpu.sync_copy(label_hbm.at[wid], lab_v)

        acc = jnp.zeros((_L,), jnp.float32)
        n_chunk = _B_PASS // _IDX_CHUNK
        for p in range(_N_PASS):
            for c in range(n_chunk):
                for g in range(_IDX_CHUNK // _L):
                    i0 = p * _B_PASS + c * _IDX_CHUNK + g * _L
                    lv = lab_v[pl.ds(i0, _L)]
                    idx_v[p * n_chunk + c, pl.ds(g * _L, _L)] = (
                        jnp.right_shift(lv, 1))

            cps = [
                pltpu.async_copy(
                    ptable_hbm.at[idx_v.at[p * n_chunk + c]],
                    cent_v.at[pl.ds(c * _IDX_CHUNK, _IDX_CHUNK)],
                    sem,
                )
                for c in range(n_chunk)
            ]
            pltpu.sync_copy(
                feat_hbm.at[pl.ds(base + p * _B_PASS, _B_PASS)], feat_v)
            for cp in cps:
                cp.wait()

            def group(g, acc, p=p):
                i0 = g * _L
                parv = lab_v[pl.ds(p * _B_PASS + i0, _L)]
                for u in range(_L):
                    off = (parv[u] & 1) * _FEAT_DIM
                    for j in range(_FEAT_DIM // _L):
                        f = feat_v[i0 + u, pl.ds(j * _L, _L)]
                        c = cent_v[i0 + u, pl.ds(off + j * _L, _L)]
                        d = f - c
                        acc = acc + d * d
                return acc

            acc = lax.fori_loop(0, _B_PASS // _L, group, acc)
        acc_v[...] = acc
        pltpu.sync_copy(acc_v, out_hbm.at[wid])

    return k(feat, label, ptable)


def _finish_body(p_ref, o_ref):
    s = jnp.sum(p_ref[...])
    o_ref[0, 0] = _LAMBDA_C / 2.0 / _BATCH * jnp.sqrt(s)


def kernel(feat, label, centers):
    label_r = label.astype(jnp.int32).reshape(_NW, _B_PER_W)
    ptable = _pack_pairs(centers.T)
    partials = _sc_partials(feat, label_r, ptable)
    loss = pl.pallas_call(
        _finish_body,
        out_shape=jax.ShapeDtypeStruct((1, 1), jnp.float32),
        out_specs=pl.BlockSpec(memory_space=pltpu.SMEM),
    )(partials)
    return loss[0, 0]


# speedup vs baseline: 5.9890x; 5.9890x over previous
import functools

import jax
import jax.numpy as jnp
from jax import lax
from jax.experimental import pallas as pl
from jax.experimental.pallas import tpu as pltpu
from jax.experimental.pallas import tpu_sc as plsc

_FEAT_DIM = 64
_NUM_CLASSES = 100000
_BATCH = 16384
_LAMBDA_C = 1.0

_NC = 2
_NS = 16
_L = 16
_NW = _NC * _NS
_B_PER_W = _BATCH // _NW
_N_PASS = 2
_B_PASS = _B_PER_W // _N_PASS
_IDX_CHUNK = 128
_C_BLK = 128
_N_BLK = (_NUM_CLASSES + _C_BLK - 1) // _C_BLK
_P_ROWS = _N_BLK * (_C_BLK // 2)


_K_SUB = 17
_N_STEP = (_N_BLK + _K_SUB - 1) // _K_SUB


def _pack_body(ct_ref, out_ref):
    q = lax.broadcasted_iota(jnp.int32, (_C_BLK, _C_BLK), 0)
    k = lax.broadcasted_iota(jnp.int32, (_C_BLK, _C_BLK), 1)
    sel = (k == 2 * (q % (_C_BLK // 2)) + q // (_C_BLK // 2)).astype(
        jnp.float32)
    dn = (((1,), (1,)), ((), ()))
    for s in range(_K_SUB):
        ct = ct_ref[:, pl.ds(s * _C_BLK, _C_BLK)]
        both = lax.dot_general(
            sel, ct, dn, preferred_element_type=jnp.float32)
        r0 = s * (_C_BLK // 2)
        out_ref[pl.ds(r0, _C_BLK // 2), 0:_FEAT_DIM] = both[0:_C_BLK // 2]
        out_ref[pl.ds(r0, _C_BLK // 2), _FEAT_DIM:] = both[_C_BLK // 2:]


def _pack_pairs(centers_t):
    return pl.pallas_call(
        _pack_body,
        grid=(_N_STEP,),
        in_specs=[pl.BlockSpec((_FEAT_DIM, _K_SUB * _C_BLK), lambda b: (0, b))],
        out_specs=pl.BlockSpec((_K_SUB * _C_BLK // 2, _C_BLK), lambda b: (b, 0)),
        out_shape=jax.ShapeDtypeStruct((_N_STEP * _K_SUB * _C_BLK // 2, _C_BLK),
                                       jnp.float32),
    )(centers_t)


def _sc_partials(feat, label, ptable):
    mesh = plsc.VectorSubcoreMesh(core_axis_name="c", subcore_axis_name="s")

    @functools.partial(
        pl.kernel,
        mesh=mesh,
        out_type=jax.ShapeDtypeStruct((_NW, _L), jnp.float32),
        scratch_types=[
            pltpu.VMEM((_B_PER_W,), jnp.int32),
            pltpu.VMEM((_B_PER_W // _IDX_CHUNK, _IDX_CHUNK), jnp.int32),
            pltpu.VMEM((_B_PASS, _FEAT_DIM), jnp.float32),
            pltpu.VMEM((_B_PASS, _C_BLK), jnp.float32),
            pltpu.VMEM((_L,), jnp.float32),
            pltpu.SemaphoreType.DMA,
        ],
        compiler_params=pltpu.CompilerParams(use_tc_tiling_on_sc=True),
    )
    def k(feat_hbm, label_hbm, ptable_hbm, out_hbm,
          lab_v, idx_v, feat_v, cent_v, acc_v, sem):
        wid = lax.axis_index("s") * _NC + lax.axis_index("c")
        base = wid * _B_PER_W
        pltpu.sync_copy(label_hbm.at[wid], lab_v)

        acc = jnp.zeros((_L,), jnp.float32)
        n_chunk = _B_PASS // _IDX_CHUNK
        for p in range(_N_PASS):
            for c in range(n_chunk):
                for g in range(_IDX_CHUNK // _L):
                    i0 = p * _B_PASS + c * _IDX_CHUNK + g * _L
                    lv = lab_v[pl.ds(i0, _L)]
                    idx_v[p * n_chunk + c, pl.ds(g * _L, _L)] = (
                        jnp.right_shift(lv, 1))

            cps = [
                pltpu.async_copy(
                    ptable_hbm.at[idx_v.at[p * n_chunk + c]],
                    cent_v.at[pl.ds(c * _IDX_CHUNK, _IDX_CHUNK)],
                    sem,
                )
                for c in range(n_chunk)
            ]
            pltpu.sync_copy(
                feat_hbm.at[pl.ds(base + p * _B_PASS, _B_PASS)], feat_v)
            for cp in cps:
                cp.wait()

            def group(g, acc, p=p):
                i0 = g * _L
                parv = lab_v[pl.ds(p * _B_PASS + i0, _L)]
                for u in range(_L):
                    off = (parv[u] & 1) * _FEAT_DIM
                    for j in range(_FEAT_DIM // _L):
                        f = feat_v[i0 + u, pl.ds(j * _L, _L)]
                        c = cent_v[i0 + u, pl.ds(off + j * _L, _L)]
                        d = f - c
                        acc = acc + d * d
                return acc

            acc = lax.fori_loop(0, _B_PASS // _L, group, acc)
        acc_v[...] = acc
        pltpu.sync_copy(acc_v, out_hbm.at[wid])

    return k(feat, label, ptable)


def _finish_body(p_ref, o_ref):
    s = jnp.sum(p_ref[...])
    o_ref[0, 0] = _LAMBDA_C / 2.0 / _BATCH * jnp.sqrt(s)


def kernel(feat, label, centers):
    label_r = label.astype(jnp.int32).reshape(_NW, _B_PER_W)
    ptable = _pack_pairs(centers.T)
    partials = _sc_partials(feat, label_r, ptable)
    loss = pl.pallas_call(
        _finish_body,
        out_shape=jax.ShapeDtypeStruct((1, 1), jnp.float32),
        out_specs=pl.BlockSpec(memory_space=pltpu.SMEM),
    )(partials)
    return loss[0, 0]


# speedup vs baseline: 7.4770x; 1.2485x over previous
import functools

import jax
import jax.numpy as jnp
from jax import lax
from jax.experimental import pallas as pl
from jax.experimental.pallas import tpu as pltpu
from jax.experimental.pallas import tpu_sc as plsc

_FEAT_DIM = 64
_NUM_CLASSES = 100000
_BATCH = 16384
_LAMBDA_C = 1.0

_NC = 2
_NS = 16
_L = 16
_NW = _NC * _NS
_B_PER_W = _BATCH // _NW
_N_PASS = 2
_B_PASS = _B_PER_W // _N_PASS
_IDX_CHUNK = 128
_C_BLK = 128
_N_BLK = (_NUM_CLASSES + _C_BLK - 1) // _C_BLK
_P_ROWS = _N_BLK * (_C_BLK // 2)


_K_SUB = 46
_N_STEP = (_N_BLK + _K_SUB - 1) // _K_SUB


def _pack_body(ct_ref, out_ref):
    q = lax.broadcasted_iota(jnp.int32, (_C_BLK, _C_BLK), 0)
    k = lax.broadcasted_iota(jnp.int32, (_C_BLK, _C_BLK), 1)
    sel = (k == 2 * (q % (_C_BLK // 2)) + q // (_C_BLK // 2)).astype(
        jnp.float32)
    dn = (((1,), (1,)), ((), ()))
    for s in range(_K_SUB):
        ct = ct_ref[:, pl.ds(s * _C_BLK, _C_BLK)]
        both = lax.dot_general(
            sel, ct, dn, preferred_element_type=jnp.float32)
        r0 = s * (_C_BLK // 2)
        out_ref[pl.ds(r0, _C_BLK // 2), 0:_FEAT_DIM] = both[0:_C_BLK // 2]
        out_ref[pl.ds(r0, _C_BLK // 2), _FEAT_DIM:] = both[_C_BLK // 2:]


def _pack_pairs(centers_t):
    return pl.pallas_call(
        _pack_body,
        grid=(_N_STEP,),
        in_specs=[pl.BlockSpec((_FEAT_DIM, _K_SUB * _C_BLK), lambda b: (0, b))],
        out_specs=pl.BlockSpec((_K_SUB * _C_BLK // 2, _C_BLK), lambda b: (b, 0)),
        out_shape=jax.ShapeDtypeStruct((_N_STEP * _K_SUB * _C_BLK // 2, _C_BLK),
                                       jnp.float32),
    )(centers_t)


def _sc_partials(feat, label, ptable):
    mesh = plsc.VectorSubcoreMesh(core_axis_name="c", subcore_axis_name="s")

    @functools.partial(
        pl.kernel,
        mesh=mesh,
        out_type=jax.ShapeDtypeStruct((_NW, _L), jnp.float32),
        scratch_types=[
            pltpu.VMEM((_B_PER_W,), jnp.int32),
            pltpu.VMEM((_B_PER_W // _IDX_CHUNK, _IDX_CHUNK), jnp.int32),
            pltpu.VMEM((_B_PASS, _FEAT_DIM), jnp.float32),
            pltpu.VMEM((_B_PASS, _C_BLK), jnp.float32),
            pltpu.VMEM((_L,), jnp.float32),
            pltpu.SemaphoreType.DMA,
        ],
        compiler_params=pltpu.CompilerParams(use_tc_tiling_on_sc=True),
    )
    def k(feat_hbm, label_hbm, ptable_hbm, out_hbm,
          lab_v, idx_v, feat_v, cent_v, acc_v, sem):
        wid = lax.axis_index("s") * _NC + lax.axis_index("c")
        base = wid * _B_PER_W
        pltpu.sync_copy(label_hbm.at[wid], lab_v)

        acc = jnp.zeros((_L,), jnp.float32)
        n_chunk = _B_PASS // _IDX_CHUNK
        for p in range(_N_PASS):
            for c in range(n_chunk):
                for g in range(_IDX_CHUNK // _L):
                    i0 = p * _B_PASS + c * _IDX_CHUNK + g * _L
                    lv = lab_v[pl.ds(i0, _L)]
                    idx_v[p * n_chunk + c, pl.ds(g * _L, _L)] = (
                        jnp.right_shift(lv, 1))

            cps = [
                pltpu.async_copy(
                    ptable_hbm.at[idx_v.at[p * n_chunk + c]],
                    cent_v.at[pl.ds(c * _IDX_CHUNK, _IDX_CHUNK)],
                    sem,
                )
                for c in range(n_chunk)
            ]
            pltpu.sync_copy(
                feat_hbm.at[pl.ds(base + p * _B_PASS, _B_PASS)], feat_v)
            for cp in cps:
                cp.wait()

            def group(g, acc, p=p):
                i0 = g * _L
                parv = lab_v[pl.ds(p * _B_PASS + i0, _L)]
                for u in range(_L):
                    off = (parv[u] & 1) * _FEAT_DIM
                    for j in range(_FEAT_DIM // _L):
                        f = feat_v[i0 + u, pl.ds(j * _L, _L)]
                        c = cent_v[i0 + u, pl.ds(off + j * _L, _L)]
                        d = f - c
                        acc = acc + d * d
                return acc

            acc = lax.fori_loop(0, _B_PASS // _L, group, acc)
        acc_v[...] = acc
        pltpu.sync_copy(acc_v, out_hbm.at[wid])

    return k(feat, label, ptable)


def _finish_body(p_ref, o_ref):
    s = jnp.sum(p_ref[...])
    o_ref[0, 0] = _LAMBDA_C / 2.0 / _BATCH * jnp.sqrt(s)


def kernel(feat, label, centers):
    label_r = label.astype(jnp.int32).reshape(_NW, _B_PER_W)
    ptable = _pack_pairs(centers.T)
    partials = _sc_partials(feat, label_r, ptable)
    loss = pl.pallas_call(
        _finish_body,
        out_shape=jax.ShapeDtypeStruct((1, 1), jnp.float32),
        out_specs=pl.BlockSpec(memory_space=pltpu.SMEM),
    )(partials)
    return loss[0, 0]


# speedup vs baseline: 7.7310x; 1.0340x over previous
import functools

import jax
import jax.numpy as jnp
from jax import lax
from jax.experimental import pallas as pl
from jax.experimental.pallas import tpu as pltpu
from jax.experimental.pallas import tpu_sc as plsc

_FEAT_DIM = 64
_NUM_CLASSES = 100000
_BATCH = 16384
_LAMBDA_C = 1.0

_NC = 2
_NS = 16
_L = 16
_NW = _NC * _NS
_B_PER_W = _BATCH // _NW
_N_PASS = 2
_B_PASS = _B_PER_W // _N_PASS
_IDX_CHUNK = 128
_C_BLK = 128
_N_BLK = (_NUM_CLASSES + _C_BLK - 1) // _C_BLK
_P_ROWS = _N_BLK * (_C_BLK // 2)


_K_SUB = 46
_N_STEP = (_N_BLK + _K_SUB - 1) // _K_SUB


def _pack_body(ct_ref, out_ref, k_sub):
    q = lax.broadcasted_iota(jnp.int32, (_C_BLK, _C_BLK), 0)
    k = lax.broadcasted_iota(jnp.int32, (_C_BLK, _C_BLK), 1)
    sel = (k == 2 * (q % (_C_BLK // 2)) + q // (_C_BLK // 2)).astype(
        jnp.float32)
    dn = (((1,), (1,)), ((), ()))
    for s in range(k_sub):
        ct = ct_ref[:, pl.ds(s * _C_BLK, _C_BLK)]
        both = lax.dot_general(
            sel, ct, dn, preferred_element_type=jnp.float32)
        r0 = s * (_C_BLK // 2)
        out_ref[pl.ds(r0, _C_BLK // 2), 0:_FEAT_DIM] = both[0:_C_BLK // 2]
        out_ref[pl.ds(r0, _C_BLK // 2), _FEAT_DIM:] = both[_C_BLK // 2:]


def _pack_pairs(x_t, k_sub, n_step):
    return pl.pallas_call(
        functools.partial(_pack_body, k_sub=k_sub),
        grid=(n_step,),
        in_specs=[pl.BlockSpec((_FEAT_DIM, k_sub * _C_BLK), lambda b: (0, b))],
        out_specs=pl.BlockSpec((k_sub * _C_BLK // 2, _C_BLK), lambda b: (b, 0)),
        out_shape=jax.ShapeDtypeStruct((n_step * k_sub * _C_BLK // 2, _C_BLK),
                                       jnp.float32),
    )(x_t)


def _sc_partials(feat, label, ptable):
    mesh = plsc.VectorSubcoreMesh(core_axis_name="c", subcore_axis_name="s")

    @functools.partial(
        pl.kernel,
        mesh=mesh,
        out_type=jax.ShapeDtypeStruct((_NW, _L), jnp.float32),
        scratch_types=[
            pltpu.VMEM((_B_PER_W,), jnp.int32),
            pltpu.VMEM((_B_PER_W // _IDX_CHUNK, _IDX_CHUNK), jnp.int32),
            pltpu.VMEM((_B_PASS // 2, _C_BLK), jnp.float32),
            pltpu.VMEM((_B_PASS, _C_BLK), jnp.float32),
            pltpu.VMEM((_L,), jnp.float32),
            pltpu.SemaphoreType.DMA,
        ],
        compiler_params=pltpu.CompilerParams(use_tc_tiling_on_sc=True),
    )
    def k(feat_hbm, label_hbm, ptable_hbm, out_hbm,
          lab_v, idx_v, feat_v, cent_v, acc_v, sem):
        wid = lax.axis_index("s") * _NC + lax.axis_index("c")
        base = wid * _B_PER_W
        pltpu.sync_copy(label_hbm.at[wid], lab_v)

        acc = jnp.zeros((_L,), jnp.float32)
        n_chunk = _B_PASS // _IDX_CHUNK
        for p in range(_N_PASS):
            for c in range(n_chunk):
                for g in range(_IDX_CHUNK // _L):
                    i0 = p * _B_PASS + c * _IDX_CHUNK + g * _L
                    lv = lab_v[pl.ds(i0, _L)]
                    idx_v[p * n_chunk + c, pl.ds(g * _L, _L)] = (
                        jnp.right_shift(lv, 1))

            cps = [
                pltpu.async_copy(
                    ptable_hbm.at[idx_v.at[p * n_chunk + c]],
                    cent_v.at[pl.ds(c * _IDX_CHUNK, _IDX_CHUNK)],
                    sem,
                )
                for c in range(n_chunk)
            ]
            fstart = pl.multiple_of(
                (base + p * _B_PASS) // 2, _C_BLK // 2)
            pltpu.sync_copy(
                feat_hbm.at[pl.ds(fstart, _B_PASS // 2)], feat_v)
            for cp in cps:
                cp.wait()

            def group(g, acc, p=p):
                i0 = g * _L
                parv = lab_v[pl.ds(p * _B_PASS + i0, _L)]
                for u in range(_L):
                    off = (parv[u] & 1) * _FEAT_DIM
                    foff = (u % 2) * _FEAT_DIM
                    for j in range(_FEAT_DIM // _L):
                        f = feat_v[g * (_L // 2) + u // 2,
                                   pl.ds(foff + j * _L, _L)]
                        c = cent_v[i0 + u, pl.ds(off + j * _L, _L)]
                        d = f - c
                        acc = acc + d * d
                return acc

            acc = lax.fori_loop(0, _B_PASS // _L, group, acc)
        acc_v[...] = acc
        pltpu.sync_copy(acc_v, out_hbm.at[wid])

    return k(feat, label, ptable)


def _finish_body(p_ref, o_ref):
    s = jnp.sum(p_ref[...])
    o_ref[0, 0] = _LAMBDA_C / 2.0 / _BATCH * jnp.sqrt(s)


def kernel(feat, label, centers):
    label_r = label.astype(jnp.int32).reshape(_NW, _B_PER_W)
    ptable = _pack_pairs(centers.T, _K_SUB, _N_STEP)
    featp = _pack_pairs(feat.T, 32, _BATCH // _C_BLK // 32)
    partials = _sc_partials(featp, label_r, ptable)
    loss = pl.pallas_call(
        _finish_body,
        out_shape=jax.ShapeDtypeStruct((1, 1), jnp.float32),
        out_specs=pl.BlockSpec(memory_space=pltpu.SMEM),
    )(partials)
    return loss[0, 0]
